# Initial kernel scaffold; baseline (speedup 1.0000x reference)
import jax
import jax.numpy as jnp
from jax.experimental import pallas as pl


def kernel(node_feats, node_attrs, edge_embedding, edge_attrs, edge_index, W1, Wm1, Wm2, W2, Wu, W3, Wsc):
    raise NotImplementedError("write your pallas kernel here")


# speedup vs baseline: 2.5277x; 2.5277x over previous
import functools
import math

import jax
import jax.numpy as jnp
from jax import lax
from jax.experimental import pallas as pl
from jax.experimental.pallas import tpu as pltpu
from jax.experimental.pallas import tpu_sc as plsc

N = 10000
E = 320000
D = 128
DA = 16
DE = 16
H = 8
AVG_NEIGH = 32.0

NC = 2
NS = 16
NW = NC * NS
CHUNK = 128
NCHUNK = E // CHUNK
_BASE_CH = NCHUNK // NW
_EXTRA = NCHUNK % NW
_MAXCH = _BASE_CH + 1
_ROWS = 624
_TAIL = N - NS * _ROWS

@functools.cache
def _sc_kernels():
    mesh = plsc.VectorSubcoreMesh(core_axis_name="c", subcore_axis_name="s",
                                  num_cores=NC, num_subcores=NS)

    @functools.partial(
        pl.kernel,
        out_type=jax.ShapeDtypeStruct((E, D), jnp.float32),
        mesh=mesh,
        scratch_types=[
            pltpu.VMEM((_MAXCH, CHUNK), jnp.int32),
            pltpu.VMEM((CHUNK, D), jnp.float32),
            pltpu.SemaphoreType.DMA,
        ],
    )
    def sc_gather(x_hbm, src_hbm, xs_hbm, idx_v, buf_v, sem):
        c = lax.axis_index("c")
        s = lax.axis_index("s")
        wid = s * NC + c
        pltpu.sync_copy(src_hbm.at[wid], idx_v)
        nj = _BASE_CH + jnp.where(wid < _EXTRA, 1, 0)

        def body(i, carry):
            j = wid + i * NW
            pltpu.async_copy(x_hbm.at[idx_v.at[i]], buf_v, sem).wait()
            pltpu.sync_copy(buf_v, xs_hbm.at[pl.ds(j * CHUNK, CHUNK)])
            return carry

        lax.fori_loop(0, nj, body, 0)

    @functools.partial(
        pl.kernel,
        out_type=jax.ShapeDtypeStruct((NC, N, D), jnp.float32),
        mesh=mesh,
        scratch_types=[
            pltpu.VMEM((_MAXCH, CHUNK), jnp.int32),
            pltpu.VMEM((CHUNK, D), jnp.float32),
            pltpu.VMEM_SHARED((N, D), jnp.float32),
        ],
    )
    def sc_scatter(msg_hbm, dst_hbm, zero_hbm, out_hbm, idx_v, buf_v, acc_sh):
        c = lax.axis_index("c")
        s = lax.axis_index("s")
        wid = s * NC + c
        pltpu.sync_copy(dst_hbm.at[wid], idx_v)
        pltpu.sync_copy(zero_hbm.at[pl.ds(s * _ROWS, _ROWS)],
                        acc_sh.at[pl.ds(s * _ROWS, _ROWS)])

        @pl.when(s == NS - 1)
        def _():
            pltpu.sync_copy(zero_hbm.at[pl.ds(NS * _ROWS, _TAIL)],
                            acc_sh.at[pl.ds(NS * _ROWS, _TAIL)])

        plsc.subcore_barrier()

        nj = _BASE_CH + jnp.where(wid < _EXTRA, 1, 0)

        def body(i, carry):
            j = wid + i * NW
            pltpu.sync_copy(msg_hbm.at[pl.ds(j * CHUNK, CHUNK)], buf_v)
            pltpu.sync_copy(buf_v, acc_sh.at[idx_v.at[i]], add=True)
            return carry

        lax.fori_loop(0, nj, body, 0)
        plsc.subcore_barrier()
        pltpu.sync_copy(acc_sh.at[pl.ds(s * _ROWS, _ROWS)],
                        out_hbm.at[c, pl.ds(s * _ROWS, _ROWS)])

        @pl.when(s == NS - 1)
        def _():
            pltpu.sync_copy(acc_sh.at[pl.ds(NS * _ROWS, _TAIL)],
                            out_hbm.at[c, pl.ds(NS * _ROWS, _TAIL)])

    return sc_gather, sc_scatter


def _x_body(nf_ref, w1_ref, o_ref):
    o_ref[...] = jnp.dot(nf_ref[...], w1_ref[...],
                         preferred_element_type=jnp.float32) * (1.0 / math.sqrt(D))


def _tc_x(node_feats, W1):
    bn = 2000
    return pl.pallas_call(
        _x_body,
        grid=(N // bn,),
        in_specs=[
            pl.BlockSpec((bn, D), lambda i: (i, 0)),
            pl.BlockSpec((D, D), lambda i: (0, 0)),
        ],
        out_specs=pl.BlockSpec((bn, D), lambda i: (i, 0)),
        out_shape=jax.ShapeDtypeStruct((N, D), jnp.float32),
    )(node_feats, W1)


def _edge_body(xs_ref, ee_ref, ea_ref, wm1_ref, wm2_ref, w2_ref, o_ref):
    h = jnp.dot(ee_ref[...], wm1_ref[...],
                preferred_element_type=jnp.float32) * (1.0 / math.sqrt(DE))
    h = jax.nn.silu(h)
    w = jnp.dot(h, wm2_ref[...],
                preferred_element_type=jnp.float32) * (1.0 / math.sqrt(H))
    z = xs_ref[...] * ea_ref[...] * w
    m = jnp.dot(z, w2_ref[...],
                preferred_element_type=jnp.float32) * (1.0 / math.sqrt(D))
    o_ref[...] = jax.nn.silu(m)


def _tc_edge(xs, edge_embedding, edge_attrs, Wm1, Wm2, W2):
    be = 2000
    return pl.pallas_call(
        _edge_body,
        grid=(E // be,),
        in_specs=[
            pl.BlockSpec((be, D), lambda i: (i, 0)),
            pl.BlockSpec((be, DE), lambda i: (i, 0)),
            pl.BlockSpec((be, 1), lambda i: (i, 0)),
            pl.BlockSpec((DE, H), lambda i: (0, 0)),
            pl.BlockSpec((H, D), lambda i: (0, 0)),
            pl.BlockSpec((D, D), lambda i: (0, 0)),
        ],
        out_specs=pl.BlockSpec((be, D), lambda i: (i, 0)),
        out_shape=jax.ShapeDtypeStruct((E, D), jnp.float32),
    )(xs, edge_embedding, edge_attrs, Wm1, Wm2, W2)


def _final_body(p_ref, nf_ref, na_ref, wut_ref, w3_ref, wsct_ref, o_ref):
    na = na_ref[...]
    u = jnp.dot(na, wut_ref[...],
                preferred_element_type=jnp.float32) * (1.0 / math.sqrt(DA))
    agg = (p_ref[0] + p_ref[1]) * (1.0 / math.sqrt(AVG_NEIGH))
    upd = jnp.dot(agg * u, w3_ref[...],
                  preferred_element_type=jnp.float32) * (1.0 / math.sqrt(D))
    nf = nf_ref[...]
    sc = jnp.zeros_like(upd)
    for v in range(DA):
        wv = wsct_ref[pl.ds(v * D, D), :]
        sc = sc + na[:, v:v + 1] * jnp.dot(nf, wv,
                                           preferred_element_type=jnp.float32)
    o_ref[...] = jax.nn.silu(upd + sc * (1.0 / math.sqrt(D * DA)))


def _tc_final(parts, node_feats, node_attrs, WuT, W3, WscT):
    bn = 2000
    return pl.pallas_call(
        _final_body,
        grid=(N // bn,),
        in_specs=[
            pl.BlockSpec((NC, bn, D), lambda i: (0, i, 0)),
            pl.BlockSpec((bn, D), lambda i: (i, 0)),
            pl.BlockSpec((bn, DA), lambda i: (i, 0)),
            pl.BlockSpec((DA, D), lambda i: (0, 0)),
            pl.BlockSpec((D, D), lambda i: (0, 0)),
            pl.BlockSpec((DA * D, D), lambda i: (0, 0)),
        ],
        out_specs=pl.BlockSpec((bn, D), lambda i: (i, 0)),
        out_shape=jax.ShapeDtypeStruct((N, D), jnp.float32),
    )(parts, node_feats, node_attrs, WuT, W3, WscT)


def kernel(node_feats, node_attrs, edge_embedding, edge_attrs, edge_index,
           W1, Wm1, Wm2, W2, Wu, W3, Wsc):
    order = jnp.minimum(
        jnp.arange(NW)[:, None] + jnp.arange(_MAXCH)[None, :] * NW,
        NCHUNK - 1)
    src = edge_index[0].reshape(NCHUNK, CHUNK)[order]
    dst = edge_index[1].reshape(NCHUNK, CHUNK)[order]
    WuT = Wu.T
    WscT = Wsc.transpose(1, 0, 2).reshape(DA * D, D)

    sc_gather, sc_scatter = _sc_kernels()
    x = _tc_x(node_feats, W1)
    xs = sc_gather(x, src)
    msg = _tc_edge(xs, edge_embedding, edge_attrs, Wm1, Wm2, W2)
    parts = sc_scatter(msg, dst, jnp.zeros((N, D), jnp.float32))
    return _tc_final(parts, node_feats, node_attrs, WuT, W3, WscT)


# speedup vs baseline: 3.5713x; 1.4129x over previous
import functools
import math

import jax
import jax.numpy as jnp
from jax import lax
from jax.experimental import pallas as pl
from jax.experimental.pallas import tpu as pltpu
from jax.experimental.pallas import tpu_sc as plsc

N = 10000
E = 320000
D = 128
DA = 16
DE = 16
H = 8
AVG_NEIGH = 32.0

NC = 2
NS = 16
NW = NC * NS
CHUNK = 128
NCHUNK = E // CHUNK
_BASE_CH = NCHUNK // NW
_EXTRA = NCHUNK % NW
_MAXCH = _BASE_CH + 1
NSLICE = 4
ES = E // NSLICE
_SCH = NCHUNK // NSLICE
_SBASE = _SCH // NW
_SEXTRA = _SCH % NW
_SMAXCH = _SBASE + 1
_ROWS = 624
_TAIL = N - NS * _ROWS

@functools.cache
def _sc_kernels():
    mesh = plsc.VectorSubcoreMesh(core_axis_name="c", subcore_axis_name="s",
                                  num_cores=NC, num_subcores=NS)

    @functools.partial(
        pl.kernel,
        out_type=jax.ShapeDtypeStruct((ES, D), jnp.float32),
        mesh=mesh,
        scratch_types=[
            pltpu.VMEM((_SMAXCH, CHUNK), jnp.int32),
            pltpu.VMEM((CHUNK, D), jnp.float32),
            pltpu.SemaphoreType.DMA,
        ],
    )
    def sc_gather(x_hbm, src_hbm, xs_hbm, idx_v, buf_v, sem):
        c = lax.axis_index("c")
        s = lax.axis_index("s")
        wid = s * NC + c
        pltpu.sync_copy(src_hbm.at[wid], idx_v)
        nj = _SBASE + jnp.where(wid < _SEXTRA, 1, 0)

        def body(i, carry):
            j = wid + i * NW
            pltpu.async_copy(x_hbm.at[idx_v.at[i]], buf_v, sem).wait()
            pltpu.sync_copy(buf_v, xs_hbm.at[pl.ds(j * CHUNK, CHUNK)])
            return carry

        lax.fori_loop(0, nj, body, 0)

    @functools.partial(
        pl.kernel,
        out_type=jax.ShapeDtypeStruct((NC, N, D), jnp.float32),
        mesh=mesh,
        scratch_types=[
            pltpu.VMEM((_SMAXCH, CHUNK), jnp.int32),
            pltpu.VMEM((CHUNK, D), jnp.float32),
            pltpu.VMEM_SHARED((N, D), jnp.float32),
        ],
    )
    def sc_scatter(m0, m1, m2, m3, dst_hbm, zero_hbm, out_hbm,
                   idx_v, buf_v, acc_sh):
        c = lax.axis_index("c")
        s = lax.axis_index("s")
        wid = s * NC + c
        pltpu.sync_copy(zero_hbm.at[pl.ds(s * _ROWS, _ROWS)],
                        acc_sh.at[pl.ds(s * _ROWS, _ROWS)])

        @pl.when(s == NS - 1)
        def _():
            pltpu.sync_copy(zero_hbm.at[pl.ds(NS * _ROWS, _TAIL)],
                            acc_sh.at[pl.ds(NS * _ROWS, _TAIL)])

        plsc.subcore_barrier()

        nj = _SBASE + jnp.where(wid < _SEXTRA, 1, 0)
        for q, mq in enumerate((m0, m1, m2, m3)):
            pltpu.sync_copy(dst_hbm.at[q, wid], idx_v)

            def body(i, carry, mq=mq):
                j = wid + i * NW
                pltpu.sync_copy(mq.at[pl.ds(j * CHUNK, CHUNK)], buf_v)
                pltpu.sync_copy(buf_v, acc_sh.at[idx_v.at[i]], add=True)
                return carry

            lax.fori_loop(0, nj, body, 0)

        plsc.subcore_barrier()
        pltpu.sync_copy(acc_sh.at[pl.ds(s * _ROWS, _ROWS)],
                        out_hbm.at[c, pl.ds(s * _ROWS, _ROWS)])

        @pl.when(s == NS - 1)
        def _():
            pltpu.sync_copy(acc_sh.at[pl.ds(NS * _ROWS, _TAIL)],
                            out_hbm.at[c, pl.ds(NS * _ROWS, _TAIL)])

    return sc_gather, sc_scatter


def _x_body(nf_ref, w1_ref, o_ref):
    o_ref[...] = jnp.dot(nf_ref[...], w1_ref[...],
                         preferred_element_type=jnp.float32) * (1.0 / math.sqrt(D))


def _tc_x(node_feats, W1):
    bn = 2000
    return pl.pallas_call(
        _x_body,
        grid=(N // bn,),
        in_specs=[
            pl.BlockSpec((bn, D), lambda i: (i, 0)),
            pl.BlockSpec((D, D), lambda i: (0, 0)),
        ],
        out_specs=pl.BlockSpec((bn, D), lambda i: (i, 0)),
        out_shape=jax.ShapeDtypeStruct((N, D), jnp.float32),
    )(node_feats, W1)


def _edge_body(xs_ref, eet_ref, eat_ref, wm1t_ref, wm2t_ref, w2_ref, o_ref):
    ht = jnp.dot(wm1t_ref[...], eet_ref[...],
                 preferred_element_type=jnp.float32) * (1.0 / math.sqrt(DE))
    ht = jax.nn.silu(ht)
    wt = jnp.dot(wm2t_ref[...], ht,
                 preferred_element_type=jnp.float32) * (1.0 / math.sqrt(H))
    wt = wt * eat_ref[...]
    wz = wt.T
    z = xs_ref[...] * wz
    m = jnp.dot(z, w2_ref[...],
                preferred_element_type=jnp.float32) * (1.0 / math.sqrt(D))
    o_ref[...] = jax.nn.silu(m)


def _tc_edge(xs, eet, eat, Wm1T, Wm2T, W2, q):
    be = 3200
    nb = ES // be
    return pl.pallas_call(
        _edge_body,
        grid=(nb,),
        in_specs=[
            pl.BlockSpec((be, D), lambda i: (i, 0)),
            pl.BlockSpec((DE, be), lambda i: (0, i + q * nb)),
            pl.BlockSpec((1, be), lambda i: (0, i + q * nb)),
            pl.BlockSpec((H, DE), lambda i: (0, 0)),
            pl.BlockSpec((D, H), lambda i: (0, 0)),
            pl.BlockSpec((D, D), lambda i: (0, 0)),
        ],
        out_specs=pl.BlockSpec((be, D), lambda i: (i, 0)),
        out_shape=jax.ShapeDtypeStruct((ES, D), jnp.float32),
    )(xs, eet, eat, Wm1T, Wm2T, W2)


def _final_body(p_ref, nf_ref, na_ref, wut_ref, w3_ref, wsct_ref, o_ref):
    na = na_ref[...]
    u = jnp.dot(na, wut_ref[...],
                preferred_element_type=jnp.float32) * (1.0 / math.sqrt(DA))
    agg = (p_ref[0] + p_ref[1]) * (1.0 / math.sqrt(AVG_NEIGH))
    upd = jnp.dot(agg * u, w3_ref[...],
                  preferred_element_type=jnp.float32) * (1.0 / math.sqrt(D))
    nf = nf_ref[...]
    sc = jnp.zeros_like(upd)
    for v in range(DA):
        wv = wsct_ref[pl.ds(v * D, D), :]
        sc = sc + na[:, v:v + 1] * jnp.dot(nf, wv,
                                           preferred_element_type=jnp.float32)
    o_ref[...] = jax.nn.silu(upd + sc * (1.0 / math.sqrt(D * DA)))


def _tc_final(parts, node_feats, node_attrs, WuT, W3, WscT):
    bn = 2000
    return pl.pallas_call(
        _final_body,
        grid=(N // bn,),
        in_specs=[
            pl.BlockSpec((NC, bn, D), lambda i: (0, i, 0)),
            pl.BlockSpec((bn, D), lambda i: (i, 0)),
            pl.BlockSpec((bn, DA), lambda i: (i, 0)),
            pl.BlockSpec((DA, D), lambda i: (0, 0)),
            pl.BlockSpec((D, D), lambda i: (0, 0)),
            pl.BlockSpec((DA * D, D), lambda i: (0, 0)),
        ],
        out_specs=pl.BlockSpec((bn, D), lambda i: (i, 0)),
        out_shape=jax.ShapeDtypeStruct((N, D), jnp.float32),
    )(parts, node_feats, node_attrs, WuT, W3, WscT)


def kernel(node_feats, node_attrs, edge_embedding, edge_attrs, edge_index,
           W1, Wm1, Wm2, W2, Wu, W3, Wsc):
    local = jnp.minimum(
        jnp.arange(NW)[:, None] + jnp.arange(_SMAXCH)[None, :] * NW,
        _SCH - 1)
    order = local[None] + (jnp.arange(NSLICE) * _SCH)[:, None, None]
    src = edge_index[0].reshape(NCHUNK, CHUNK)[order]
    dst = edge_index[1].reshape(NCHUNK, CHUNK)[order]
    eet = edge_embedding.T
    eat = edge_attrs.T
    Wm1T = Wm1.T
    Wm2T = Wm2.T
    WuT = Wu.T
    WscT = Wsc.transpose(1, 0, 2).reshape(DA * D, D)

    sc_gather, sc_scatter = _sc_kernels()
    x = _tc_x(node_feats, W1)
    msgs = []
    for q in range(NSLICE):
        xs_q = sc_gather(x, src[q])
        msgs.append(_tc_edge(xs_q, eet, eat, Wm1T, Wm2T, W2, q))
    parts = sc_scatter(*msgs, dst, jnp.zeros((N, D), jnp.float32))
    return _tc_final(parts, node_feats, node_attrs, WuT, W3, WscT)


# speedup vs baseline: 4.1720x; 1.1682x over previous
import functools
import math

import jax
import jax.numpy as jnp
from jax import lax
from jax.experimental import pallas as pl
from jax.experimental.pallas import tpu as pltpu
from jax.experimental.pallas import tpu_sc as plsc

N = 10000
E = 320000
D = 128
DA = 16
DE = 16
H = 8
AVG_NEIGH = 32.0

NC = 2
NS = 16
NW = NC * NS
CHUNK = 128
NCHUNK = E // CHUNK
_BASE_CH = NCHUNK // NW
_EXTRA = NCHUNK % NW
_MAXCH = _BASE_CH + 1
NSLICE = 4
ES = E // NSLICE
_SCH = NCHUNK // NSLICE
_SBASE = _SCH // NW
_SEXTRA = _SCH % NW
_SMAXCH = _SBASE + 1
_ROWS = 624
_TAIL = N - NS * _ROWS

@functools.cache
def _sc_kernels():
    mesh = plsc.VectorSubcoreMesh(core_axis_name="c", subcore_axis_name="s",
                                  num_cores=NC, num_subcores=NS)

    @functools.partial(
        pl.kernel,
        out_type=jax.ShapeDtypeStruct((ES, D), jnp.float32),
        mesh=mesh,
        scratch_types=[
            pltpu.VMEM((_SMAXCH, CHUNK), jnp.int32),
            pltpu.VMEM((3, CHUNK, D), jnp.float32),
            pltpu.SemaphoreType.DMA((3,)),
            pltpu.SemaphoreType.DMA((3,)),
        ],
    )
    def sc_gather(x_hbm, src_hbm, xs_hbm, idx_v, buf_v, sem_g, sem_w):
        c = lax.axis_index("c")
        s = lax.axis_index("s")
        wid = s * NC + c
        pltpu.sync_copy(src_hbm.at[wid], idx_v)
        nj = _SBASE + jnp.where(wid < _SEXTRA, 1, 0)
        pltpu.async_copy(x_hbm.at[idx_v.at[0]], buf_v.at[0], sem_g.at[0])

        def body(i, carry):
            b = i % 3
            j = wid + i * NW
            pltpu.make_async_copy(x_hbm.at[idx_v.at[i]], buf_v.at[b],
                                  sem_g.at[b]).wait()
            pltpu.async_copy(buf_v.at[b], xs_hbm.at[pl.ds(j * CHUNK, CHUNK)],
                             sem_w.at[b])

            @pl.when(i + 1 < nj)
            def _():
                bn = (i + 1) % 3

                @pl.when(i + 1 >= 3)
                def _():
                    pltpu.make_async_copy(
                        buf_v.at[bn],
                        xs_hbm.at[pl.ds((wid + (i - 2) * NW) * CHUNK, CHUNK)],
                        sem_w.at[bn]).wait()

                pltpu.async_copy(x_hbm.at[idx_v.at[i + 1]], buf_v.at[bn],
                                 sem_g.at[bn])

            return carry

        lax.fori_loop(0, nj, body, 0)
        for k in (3, 2, 1):
            b = (nj - k) % 3
            pltpu.make_async_copy(
                buf_v.at[b],
                xs_hbm.at[pl.ds((wid + (nj - k) * NW) * CHUNK, CHUNK)],
                sem_w.at[b]).wait()

    @functools.partial(
        pl.kernel,
        out_type=jax.ShapeDtypeStruct((NC, N, D), jnp.float32),
        mesh=mesh,
        scratch_types=[
            pltpu.VMEM((_SMAXCH, CHUNK), jnp.int32),
            pltpu.VMEM((_SMAXCH, CHUNK), jnp.int32),
            pltpu.VMEM((2, CHUNK, D), jnp.float32),
            pltpu.VMEM_SHARED((N, D), jnp.float32),
            pltpu.SemaphoreType.DMA((2,)),
            pltpu.SemaphoreType.DMA((2,)),
        ],
    )
    def sc_scatter(m0, m1, dst_hbm, zero_hbm, out_hbm,
                   idx0_v, idx1_v, buf_v, acc_sh, sem_r, sem_a):
        c = lax.axis_index("c")
        s = lax.axis_index("s")
        wid = s * NC + c
        pltpu.sync_copy(zero_hbm.at[pl.ds(s * _ROWS, _ROWS)],
                        acc_sh.at[pl.ds(s * _ROWS, _ROWS)])

        @pl.when(s == NS - 1)
        def _():
            pltpu.sync_copy(zero_hbm.at[pl.ds(NS * _ROWS, _TAIL)],
                            acc_sh.at[pl.ds(NS * _ROWS, _TAIL)])

        plsc.subcore_barrier()

        nj = _SBASE + jnp.where(wid < _SEXTRA, 1, 0)
        for q, (mq, idx_v) in enumerate(((m0, idx0_v), (m1, idx1_v))):
            pltpu.sync_copy(dst_hbm.at[q, wid], idx_v)
            pltpu.async_copy(mq.at[pl.ds(wid * CHUNK, CHUNK)], buf_v.at[0],
                             sem_r.at[0])

            def body(i, carry, mq=mq, idx_v=idx_v):
                b = i % 2
                pltpu.make_async_copy(
                    mq.at[pl.ds((wid + i * NW) * CHUNK, CHUNK)],
                    buf_v.at[b], sem_r.at[b]).wait()
                pltpu.async_copy(buf_v.at[b], acc_sh.at[idx_v.at[i]],
                                 sem_a.at[b], add=True)

                @pl.when(i + 1 < nj)
                def _():
                    bn = (i + 1) % 2

                    @pl.when(i + 1 >= 2)
                    def _():
                        pltpu.make_async_copy(
                            buf_v.at[bn], acc_sh.at[idx_v.at[i - 1]],
                            sem_a.at[bn]).wait()

                    pltpu.async_copy(
                        mq.at[pl.ds((wid + (i + 1) * NW) * CHUNK, CHUNK)],
                        buf_v.at[bn], sem_r.at[bn])

                return carry

            lax.fori_loop(0, nj, body, 0)
            for k in (2, 1):
                b = (nj - k) % 2
                pltpu.make_async_copy(buf_v.at[b],
                                      acc_sh.at[idx_v.at[nj - k]],
                                      sem_a.at[b]).wait()

        plsc.subcore_barrier()
        pltpu.sync_copy(acc_sh.at[pl.ds(s * _ROWS, _ROWS)],
                        out_hbm.at[c, pl.ds(s * _ROWS, _ROWS)])

        @pl.when(s == NS - 1)
        def _():
            pltpu.sync_copy(acc_sh.at[pl.ds(NS * _ROWS, _TAIL)],
                            out_hbm.at[c, pl.ds(NS * _ROWS, _TAIL)])

    return sc_gather, sc_scatter


def _x_body(nf_ref, w1_ref, o_ref):
    o_ref[...] = jnp.dot(nf_ref[...], w1_ref[...],
                         preferred_element_type=jnp.float32) * (1.0 / math.sqrt(D))


def _tc_x(node_feats, W1):
    bn = 2000
    return pl.pallas_call(
        _x_body,
        grid=(N // bn,),
        in_specs=[
            pl.BlockSpec((bn, D), lambda i: (i, 0)),
            pl.BlockSpec((D, D), lambda i: (0, 0)),
        ],
        out_specs=pl.BlockSpec((bn, D), lambda i: (i, 0)),
        out_shape=jax.ShapeDtypeStruct((N, D), jnp.float32),
    )(node_feats, W1)


def _edge_body(xs_ref, eet_ref, eat_ref, wm1t_ref, wm2t_ref, w2_ref, o_ref):
    ht = jnp.dot(wm1t_ref[...], eet_ref[...],
                 preferred_element_type=jnp.float32) * (1.0 / math.sqrt(DE))
    ht = jax.nn.silu(ht)
    wt = jnp.dot(wm2t_ref[...], ht,
                 preferred_element_type=jnp.float32) * (1.0 / math.sqrt(H))
    wt = wt * eat_ref[...]
    wz = wt.T
    z = xs_ref[...] * wz
    m = jnp.dot(z, w2_ref[...],
                preferred_element_type=jnp.float32) * (1.0 / math.sqrt(D))
    o_ref[...] = jax.nn.silu(m)


def _tc_edge(xs, eet, eat, Wm1T, Wm2T, W2, q):
    be = 3200
    nb = ES // be
    return pl.pallas_call(
        _edge_body,
        grid=(nb,),
        in_specs=[
            pl.BlockSpec((be, D), lambda i: (i, 0)),
            pl.BlockSpec((DE, be), lambda i: (0, i + q * nb)),
            pl.BlockSpec((1, be), lambda i: (0, i + q * nb)),
            pl.BlockSpec((H, DE), lambda i: (0, 0)),
            pl.BlockSpec((D, H), lambda i: (0, 0)),
            pl.BlockSpec((D, D), lambda i: (0, 0)),
        ],
        out_specs=pl.BlockSpec((be, D), lambda i: (i, 0)),
        out_shape=jax.ShapeDtypeStruct((ES, D), jnp.float32),
    )(xs, eet, eat, Wm1T, Wm2T, W2)


def _final_body(pa_ref, pb_ref, nf_ref, na_ref, wut_ref, w3_ref, wsct_ref,
                o_ref):
    na = na_ref[...]
    u = jnp.dot(na, wut_ref[...],
                preferred_element_type=jnp.float32) * (1.0 / math.sqrt(DA))
    agg = (pa_ref[0] + pa_ref[1] + pb_ref[0] + pb_ref[1]) * (
        1.0 / math.sqrt(AVG_NEIGH))
    upd = jnp.dot(agg * u, w3_ref[...],
                  preferred_element_type=jnp.float32) * (1.0 / math.sqrt(D))
    nf = nf_ref[...]
    sc = jnp.zeros_like(upd)
    for v in range(DA):
        wv = wsct_ref[pl.ds(v * D, D), :]
        sc = sc + na[:, v:v + 1] * jnp.dot(nf, wv,
                                           preferred_element_type=jnp.float32)
    o_ref[...] = jax.nn.silu(upd + sc * (1.0 / math.sqrt(D * DA)))


def _tc_final(partsA, partsB, node_feats, node_attrs, WuT, W3, WscT):
    bn = 2000
    return pl.pallas_call(
        _final_body,
        grid=(N // bn,),
        in_specs=[
            pl.BlockSpec((NC, bn, D), lambda i: (0, i, 0)),
            pl.BlockSpec((NC, bn, D), lambda i: (0, i, 0)),
            pl.BlockSpec((bn, D), lambda i: (i, 0)),
            pl.BlockSpec((bn, DA), lambda i: (i, 0)),
            pl.BlockSpec((DA, D), lambda i: (0, 0)),
            pl.BlockSpec((D, D), lambda i: (0, 0)),
            pl.BlockSpec((DA * D, D), lambda i: (0, 0)),
        ],
        out_specs=pl.BlockSpec((bn, D), lambda i: (i, 0)),
        out_shape=jax.ShapeDtypeStruct((N, D), jnp.float32),
    )(partsA, partsB, node_feats, node_attrs, WuT, W3, WscT)


def kernel(node_feats, node_attrs, edge_embedding, edge_attrs, edge_index,
           W1, Wm1, Wm2, W2, Wu, W3, Wsc):
    local = jnp.minimum(
        jnp.arange(NW)[:, None] + jnp.arange(_SMAXCH)[None, :] * NW,
        _SCH - 1)
    order = local[None] + (jnp.arange(NSLICE) * _SCH)[:, None, None]
    src = edge_index[0].reshape(NCHUNK, CHUNK)[order]
    dst = edge_index[1].reshape(NCHUNK, CHUNK)[order]
    eet = edge_embedding.T
    eat = edge_attrs.T
    Wm1T = Wm1.T
    Wm2T = Wm2.T
    WuT = Wu.T
    WscT = Wsc.transpose(1, 0, 2).reshape(DA * D, D)

    sc_gather, sc_scatter = _sc_kernels()
    zeros = jnp.zeros((N, D), jnp.float32)
    x = _tc_x(node_feats, W1)
    msgs = []
    partsA = None
    for q in range(NSLICE):
        xs_q = sc_gather(x, src[q])
        msgs.append(_tc_edge(xs_q, eet, eat, Wm1T, Wm2T, W2, q))
        if q == 1:
            partsA = sc_scatter(msgs[0], msgs[1], dst[0:2], zeros)
    partsB = sc_scatter(msgs[2], msgs[3], dst[2:4], zeros)
    return _tc_final(partsA, partsB, node_feats, node_attrs, WuT, W3, WscT)


# speedup vs baseline: 4.9205x; 1.1794x over previous
import functools
import math

import jax
import jax.numpy as jnp
from jax import lax
from jax.experimental import pallas as pl
from jax.experimental.pallas import tpu as pltpu
from jax.experimental.pallas import tpu_sc as plsc

N = 10000
E = 320000
D = 128
DA = 16
DE = 16
H = 8
AVG_NEIGH = 32.0

NC = 2
NS = 16
NW = NC * NS
CHUNK = 128
NCHUNK = E // CHUNK
_BASE_CH = NCHUNK // NW
_EXTRA = NCHUNK % NW
_MAXCH = _BASE_CH + 1
NSLICE = 4
ES = E // NSLICE
_SCH = NCHUNK // NSLICE
_SBASE = _SCH // NW
_SEXTRA = _SCH % NW
_SMAXCH = _SBASE + 1
_ROWS = 624
_TAIL = N - NS * _ROWS

@functools.cache
def _sc_kernels():
    mesh = plsc.VectorSubcoreMesh(core_axis_name="c", subcore_axis_name="s",
                                  num_cores=NC, num_subcores=NS)

    @functools.partial(
        pl.kernel,
        out_type=jax.ShapeDtypeStruct((ES, D), jnp.float32),
        mesh=mesh,
        scratch_types=[
            pltpu.VMEM((_SMAXCH, CHUNK), jnp.int32),
            pltpu.VMEM((2, CHUNK, D), jnp.float32),
            pltpu.VMEM_SHARED((N, D), jnp.float32),
            pltpu.SemaphoreType.DMA((2,)),
            pltpu.SemaphoreType.DMA((2,)),
        ],
    )
    def sc_gather(x_hbm, src_hbm, xs_hbm, idx_v, buf_v, x_sh, sem_g, sem_w):
        c = lax.axis_index("c")
        s = lax.axis_index("s")
        wid = s * NC + c
        pltpu.sync_copy(src_hbm.at[wid], idx_v)
        pltpu.sync_copy(x_hbm.at[pl.ds(s * _ROWS, _ROWS)],
                        x_sh.at[pl.ds(s * _ROWS, _ROWS)])

        @pl.when(s == NS - 1)
        def _():
            pltpu.sync_copy(x_hbm.at[pl.ds(NS * _ROWS, _TAIL)],
                            x_sh.at[pl.ds(NS * _ROWS, _TAIL)])

        plsc.subcore_barrier()
        nj = _SBASE + jnp.where(wid < _SEXTRA, 1, 0)
        pltpu.async_copy(x_sh.at[idx_v.at[0]], buf_v.at[0], sem_g.at[0])

        def body(i, carry):
            b = i % 2
            j = wid + i * NW
            pltpu.make_async_copy(x_sh.at[idx_v.at[i]], buf_v.at[b],
                                  sem_g.at[b]).wait()
            pltpu.async_copy(buf_v.at[b], xs_hbm.at[pl.ds(j * CHUNK, CHUNK)],
                             sem_w.at[b])

            @pl.when(i + 1 < nj)
            def _():
                bn = (i + 1) % 2

                @pl.when(i + 1 >= 2)
                def _():
                    pltpu.make_async_copy(
                        buf_v.at[bn],
                        xs_hbm.at[pl.ds((wid + (i - 1) * NW) * CHUNK, CHUNK)],
                        sem_w.at[bn]).wait()

                pltpu.async_copy(x_sh.at[idx_v.at[i + 1]], buf_v.at[bn],
                                 sem_g.at[bn])

            return carry

        lax.fori_loop(0, nj, body, 0)
        for k in (2, 1):
            b = (nj - k) % 2
            pltpu.make_async_copy(
                buf_v.at[b],
                xs_hbm.at[pl.ds((wid + (nj - k) * NW) * CHUNK, CHUNK)],
                sem_w.at[b]).wait()

    @functools.partial(
        pl.kernel,
        out_type=jax.ShapeDtypeStruct((NC, N, D), jnp.float32),
        mesh=mesh,
        scratch_types=[
            pltpu.VMEM((_SMAXCH, CHUNK), jnp.int32),
            pltpu.VMEM((_SMAXCH, CHUNK), jnp.int32),
            pltpu.VMEM((2, CHUNK, D), jnp.float32),
            pltpu.VMEM_SHARED((N, D), jnp.float32),
            pltpu.SemaphoreType.DMA((2,)),
            pltpu.SemaphoreType.DMA((2,)),
        ],
    )
    def sc_scatter(m0, m1, dst_hbm, zero_hbm, out_hbm,
                   idx0_v, idx1_v, buf_v, acc_sh, sem_r, sem_a):
        c = lax.axis_index("c")
        s = lax.axis_index("s")
        wid = s * NC + c
        pltpu.sync_copy(zero_hbm.at[pl.ds(s * _ROWS, _ROWS)],
                        acc_sh.at[pl.ds(s * _ROWS, _ROWS)])

        @pl.when(s == NS - 1)
        def _():
            pltpu.sync_copy(zero_hbm.at[pl.ds(NS * _ROWS, _TAIL)],
                            acc_sh.at[pl.ds(NS * _ROWS, _TAIL)])

        plsc.subcore_barrier()

        nj = _SBASE + jnp.where(wid < _SEXTRA, 1, 0)
        for q, (mq, idx_v) in enumerate(((m0, idx0_v), (m1, idx1_v))):
            pltpu.sync_copy(dst_hbm.at[q, wid], idx_v)
            pltpu.async_copy(mq.at[pl.ds(wid * CHUNK, CHUNK)], buf_v.at[0],
                             sem_r.at[0])

            def body(i, carry, mq=mq, idx_v=idx_v):
                b = i % 2
                pltpu.make_async_copy(
                    mq.at[pl.ds((wid + i * NW) * CHUNK, CHUNK)],
                    buf_v.at[b], sem_r.at[b]).wait()
                pltpu.async_copy(buf_v.at[b], acc_sh.at[idx_v.at[i]],
                                 sem_a.at[b], add=True)

                @pl.when(i + 1 < nj)
                def _():
                    bn = (i + 1) % 2

                    @pl.when(i + 1 >= 2)
                    def _():
                        pltpu.make_async_copy(
                            buf_v.at[bn], acc_sh.at[idx_v.at[i - 1]],
                            sem_a.at[bn]).wait()

                    pltpu.async_copy(
                        mq.at[pl.ds((wid + (i + 1) * NW) * CHUNK, CHUNK)],
                        buf_v.at[bn], sem_r.at[bn])

                return carry

            lax.fori_loop(0, nj, body, 0)
            for k in (2, 1):
                b = (nj - k) % 2
                pltpu.make_async_copy(buf_v.at[b],
                                      acc_sh.at[idx_v.at[nj - k]],
                                      sem_a.at[b]).wait()

        plsc.subcore_barrier()
        pltpu.sync_copy(acc_sh.at[pl.ds(s * _ROWS, _ROWS)],
                        out_hbm.at[c, pl.ds(s * _ROWS, _ROWS)])

        @pl.when(s == NS - 1)
        def _():
            pltpu.sync_copy(acc_sh.at[pl.ds(NS * _ROWS, _TAIL)],
                            out_hbm.at[c, pl.ds(NS * _ROWS, _TAIL)])

    return sc_gather, sc_scatter


def _x_body(nf_ref, w1_ref, o_ref):
    o_ref[...] = jnp.dot(nf_ref[...], w1_ref[...],
                         preferred_element_type=jnp.float32) * (1.0 / math.sqrt(D))


def _tc_x(node_feats, W1):
    bn = 2000
    return pl.pallas_call(
        _x_body,
        grid=(N // bn,),
        in_specs=[
            pl.BlockSpec((bn, D), lambda i: (i, 0)),
            pl.BlockSpec((D, D), lambda i: (0, 0)),
        ],
        out_specs=pl.BlockSpec((bn, D), lambda i: (i, 0)),
        out_shape=jax.ShapeDtypeStruct((N, D), jnp.float32),
    )(node_feats, W1)


def _edge_body(xs_ref, eet_ref, eat_ref, wm1t_ref, wm2t_ref, w2_ref, o_ref):
    ht = jnp.dot(wm1t_ref[...], eet_ref[...],
                 preferred_element_type=jnp.float32) * (1.0 / math.sqrt(DE))
    ht = jax.nn.silu(ht)
    wt = jnp.dot(wm2t_ref[...], ht,
                 preferred_element_type=jnp.float32) * (1.0 / math.sqrt(H))
    wt = wt * eat_ref[...]
    wz = wt.T
    z = xs_ref[...] * wz
    m = jnp.dot(z, w2_ref[...],
                preferred_element_type=jnp.float32) * (1.0 / math.sqrt(D))
    o_ref[...] = jax.nn.silu(m)


def _tc_edge(xs, eet, eat, Wm1T, Wm2T, W2, q):
    be = 3200
    nb = ES // be
    return pl.pallas_call(
        _edge_body,
        grid=(nb,),
        in_specs=[
            pl.BlockSpec((be, D), lambda i: (i, 0)),
            pl.BlockSpec((DE, be), lambda i: (0, i + q * nb)),
            pl.BlockSpec((1, be), lambda i: (0, i + q * nb)),
            pl.BlockSpec((H, DE), lambda i: (0, 0)),
            pl.BlockSpec((D, H), lambda i: (0, 0)),
            pl.BlockSpec((D, D), lambda i: (0, 0)),
        ],
        out_specs=pl.BlockSpec((be, D), lambda i: (i, 0)),
        out_shape=jax.ShapeDtypeStruct((ES, D), jnp.float32),
    )(xs, eet, eat, Wm1T, Wm2T, W2)


def _final_body(pa_ref, pb_ref, nf_ref, na_ref, wut_ref, w3_ref, wsct_ref,
                o_ref):
    na = na_ref[...]
    u = jnp.dot(na, wut_ref[...],
                preferred_element_type=jnp.float32) * (1.0 / math.sqrt(DA))
    agg = (pa_ref[0] + pa_ref[1] + pb_ref[0] + pb_ref[1]) * (
        1.0 / math.sqrt(AVG_NEIGH))
    upd = jnp.dot(agg * u, w3_ref[...],
                  preferred_element_type=jnp.float32) * (1.0 / math.sqrt(D))
    nf = nf_ref[...]
    sc = jnp.zeros_like(upd)
    for v in range(DA):
        wv = wsct_ref[pl.ds(v * D, D), :]
        sc = sc + na[:, v:v + 1] * jnp.dot(nf, wv,
                                           preferred_element_type=jnp.float32)
    o_ref[...] = jax.nn.silu(upd + sc * (1.0 / math.sqrt(D * DA)))


def _tc_final(partsA, partsB, node_feats, node_attrs, WuT, W3, WscT):
    bn = 2000
    return pl.pallas_call(
        _final_body,
        grid=(N // bn,),
        in_specs=[
            pl.BlockSpec((NC, bn, D), lambda i: (0, i, 0)),
            pl.BlockSpec((NC, bn, D), lambda i: (0, i, 0)),
            pl.BlockSpec((bn, D), lambda i: (i, 0)),
            pl.BlockSpec((bn, DA), lambda i: (i, 0)),
            pl.BlockSpec((DA, D), lambda i: (0, 0)),
            pl.BlockSpec((D, D), lambda i: (0, 0)),
            pl.BlockSpec((DA * D, D), lambda i: (0, 0)),
        ],
        out_specs=pl.BlockSpec((bn, D), lambda i: (i, 0)),
        out_shape=jax.ShapeDtypeStruct((N, D), jnp.float32),
    )(partsA, partsB, node_feats, node_attrs, WuT, W3, WscT)


def kernel(node_feats, node_attrs, edge_embedding, edge_attrs, edge_index,
           W1, Wm1, Wm2, W2, Wu, W3, Wsc):
    local = jnp.minimum(
        jnp.arange(NW)[:, None] + jnp.arange(_SMAXCH)[None, :] * NW,
        _SCH - 1)
    order = local[None] + (jnp.arange(NSLICE) * _SCH)[:, None, None]
    src = edge_index[0].reshape(NCHUNK, CHUNK)[order]
    dst = edge_index[1].reshape(NCHUNK, CHUNK)[order]
    eet = edge_embedding.T
    eat = edge_attrs.T
    Wm1T = Wm1.T
    Wm2T = Wm2.T
    WuT = Wu.T
    WscT = Wsc.transpose(1, 0, 2).reshape(DA * D, D)

    sc_gather, sc_scatter = _sc_kernels()
    zeros = jnp.zeros((N, D), jnp.float32)
    x = _tc_x(node_feats, W1)
    msgs = []
    partsA = None
    for q in range(NSLICE):
        xs_q = sc_gather(x, src[q])
        msgs.append(_tc_edge(xs_q, eet, eat, Wm1T, Wm2T, W2, q))
        if q == 1:
            partsA = sc_scatter(msgs[0], msgs[1], dst[0:2], zeros)
    partsB = sc_scatter(msgs[2], msgs[3], dst[2:4], zeros)
    return _tc_final(partsA, partsB, node_feats, node_attrs, WuT, W3, WscT)


# speedup vs baseline: 5.3838x; 1.0942x over previous
import functools
import math

import jax
import jax.numpy as jnp
from jax import lax
from jax.experimental import pallas as pl
from jax.experimental.pallas import tpu as pltpu
from jax.experimental.pallas import tpu_sc as plsc

N = 10000
E = 320000
D = 128
DA = 16
DE = 16
H = 8
AVG_NEIGH = 32.0

NC = 2
NS = 16
NW = NC * NS
CHUNK = 128
NCHUNK = E // CHUNK
_BASE_CH = NCHUNK // NW
_EXTRA = NCHUNK % NW
_MAXCH = _BASE_CH + 1
NSLICE = 4
ES = E // NSLICE
_SCH = NCHUNK // NSLICE
_SBASE = _SCH // NW
_SEXTRA = _SCH % NW
_SMAXCH = _SBASE + 1
_ROWS = 624
_TAIL = N - NS * _ROWS

@functools.cache
def _sc_kernels():
    mesh = plsc.VectorSubcoreMesh(core_axis_name="c", subcore_axis_name="s",
                                  num_cores=NC, num_subcores=NS)

    def make_gather(q):
        soff = q * _SCH

        @functools.partial(
            pl.kernel,
            out_type=jax.ShapeDtypeStruct((ES, D), jnp.float32),
            mesh=mesh,
            scratch_types=[
                pltpu.VMEM((_SMAXCH, 1, CHUNK), jnp.int32),
                pltpu.VMEM((2, CHUNK, D), jnp.float32),
                pltpu.VMEM_SHARED((N, D), jnp.float32),
                pltpu.SemaphoreType.DMA((2,)),
                pltpu.SemaphoreType.DMA((2,)),
            ],
        )
        def sc_gather(x_hbm, src_hbm, xs_hbm, idx_v, buf_v, x_sh,
                      sem_g, sem_w):
            c = lax.axis_index("c")
            s = lax.axis_index("s")
            wid = s * NC + c
            start = wid * _SBASE + jnp.minimum(wid, _SEXTRA)
            nj = _SBASE + jnp.where(wid < _SEXTRA, 1, 0)
            pltpu.sync_copy(src_hbm.at[pl.ds(soff + start, _SMAXCH)], idx_v)
            pltpu.sync_copy(x_hbm.at[pl.ds(s * _ROWS, _ROWS)],
                            x_sh.at[pl.ds(s * _ROWS, _ROWS)])

            @pl.when(s == NS - 1)
            def _():
                pltpu.sync_copy(x_hbm.at[pl.ds(NS * _ROWS, _TAIL)],
                                x_sh.at[pl.ds(NS * _ROWS, _TAIL)])

            plsc.subcore_barrier()
            pltpu.async_copy(x_sh.at[idx_v.at[0, 0]], buf_v.at[0], sem_g.at[0])

            def body(i, carry):
                b = i % 2
                l = start + i
                pltpu.make_async_copy(x_sh.at[idx_v.at[i, 0]], buf_v.at[b],
                                      sem_g.at[b]).wait()
                pltpu.async_copy(buf_v.at[b],
                                 xs_hbm.at[pl.ds(l * CHUNK, CHUNK)],
                                 sem_w.at[b])

                @pl.when(i + 1 < nj)
                def _():
                    bn = (i + 1) % 2

                    @pl.when(i + 1 >= 2)
                    def _():
                        pltpu.make_async_copy(
                            buf_v.at[bn],
                            xs_hbm.at[pl.ds((start + i - 1) * CHUNK, CHUNK)],
                            sem_w.at[bn]).wait()

                    pltpu.async_copy(x_sh.at[idx_v.at[i + 1, 0]], buf_v.at[bn],
                                     sem_g.at[bn])

                return carry

            lax.fori_loop(0, nj, body, 0)
            for k in (2, 1):
                b = (nj - k) % 2
                pltpu.make_async_copy(
                    buf_v.at[b],
                    xs_hbm.at[pl.ds((start + nj - k) * CHUNK, CHUNK)],
                    sem_w.at[b]).wait()

        return sc_gather

    def make_scatter(q0):
        @functools.partial(
            pl.kernel,
            out_type=jax.ShapeDtypeStruct((NC, N, D), jnp.float32),
            mesh=mesh,
            scratch_types=[
                pltpu.VMEM((_SMAXCH, 1, CHUNK), jnp.int32),
                pltpu.VMEM((_SMAXCH, 1, CHUNK), jnp.int32),
                pltpu.VMEM((2, CHUNK, D), jnp.float32),
                pltpu.VMEM_SHARED((N, D), jnp.float32),
                pltpu.SemaphoreType.DMA((2,)),
                pltpu.SemaphoreType.DMA((2,)),
            ],
        )
        def sc_scatter(m0, m1, dst_hbm, zero_hbm, out_hbm,
                       idx0_v, idx1_v, buf_v, acc_sh, sem_r, sem_a):
            c = lax.axis_index("c")
            s = lax.axis_index("s")
            wid = s * NC + c
            start = wid * _SBASE + jnp.minimum(wid, _SEXTRA)
            nj = _SBASE + jnp.where(wid < _SEXTRA, 1, 0)
            pltpu.sync_copy(
                dst_hbm.at[pl.ds(q0 * _SCH + start, _SMAXCH)], idx0_v)
            pltpu.sync_copy(
                dst_hbm.at[pl.ds((q0 + 1) * _SCH + start, _SMAXCH)], idx1_v)
            pltpu.sync_copy(zero_hbm.at[pl.ds(s * _ROWS, _ROWS)],
                            acc_sh.at[pl.ds(s * _ROWS, _ROWS)])

            @pl.when(s == NS - 1)
            def _():
                pltpu.sync_copy(zero_hbm.at[pl.ds(NS * _ROWS, _TAIL)],
                                acc_sh.at[pl.ds(NS * _ROWS, _TAIL)])

            plsc.subcore_barrier()

            for mq, idx_v in ((m0, idx0_v), (m1, idx1_v)):
                pltpu.async_copy(mq.at[pl.ds(start * CHUNK, CHUNK)],
                                 buf_v.at[0], sem_r.at[0])

                def body(i, carry, mq=mq, idx_v=idx_v):
                    b = i % 2
                    pltpu.make_async_copy(
                        mq.at[pl.ds((start + i) * CHUNK, CHUNK)],
                        buf_v.at[b], sem_r.at[b]).wait()
                    pltpu.async_copy(buf_v.at[b], acc_sh.at[idx_v.at[i, 0]],
                                     sem_a.at[b], add=True)

                    @pl.when(i + 1 < nj)
                    def _():
                        bn = (i + 1) % 2

                        @pl.when(i + 1 >= 2)
                        def _():
                            pltpu.make_async_copy(
                                buf_v.at[bn], acc_sh.at[idx_v.at[i - 1, 0]],
                                sem_a.at[bn]).wait()

                        pltpu.async_copy(
                            mq.at[pl.ds((start + i + 1) * CHUNK, CHUNK)],
                            buf_v.at[bn], sem_r.at[bn])

                    return carry

                lax.fori_loop(0, nj, body, 0)
                for k in (2, 1):
                    b = (nj - k) % 2
                    pltpu.make_async_copy(buf_v.at[b],
                                          acc_sh.at[idx_v.at[nj - k, 0]],
                                          sem_a.at[b]).wait()

            plsc.subcore_barrier()
            pltpu.sync_copy(acc_sh.at[pl.ds(s * _ROWS, _ROWS)],
                            out_hbm.at[c, pl.ds(s * _ROWS, _ROWS)])

            @pl.when(s == NS - 1)
            def _():
                pltpu.sync_copy(acc_sh.at[pl.ds(NS * _ROWS, _TAIL)],
                                out_hbm.at[c, pl.ds(NS * _ROWS, _TAIL)])

        return sc_scatter

    gathers = tuple(make_gather(q) for q in range(NSLICE))
    scatters = (make_scatter(0), make_scatter(2))
    return gathers, scatters


def _x_body(nf_ref, w1_ref, o_ref):
    o_ref[...] = jnp.dot(nf_ref[...], w1_ref[...],
                         preferred_element_type=jnp.float32) * (1.0 / math.sqrt(D))


def _tc_x(node_feats, W1):
    bn = 2000
    return pl.pallas_call(
        _x_body,
        grid=(N // bn,),
        in_specs=[
            pl.BlockSpec((bn, D), lambda i: (i, 0)),
            pl.BlockSpec((D, D), lambda i: (0, 0)),
        ],
        out_specs=pl.BlockSpec((bn, D), lambda i: (i, 0)),
        out_shape=jax.ShapeDtypeStruct((N, D), jnp.float32),
    )(node_feats, W1)


def _edge_body(xs_ref, eet_ref, eat_ref, wm1t_ref, wm2t_ref, w2_ref, o_ref):
    ht = jnp.dot(wm1t_ref[...], eet_ref[...],
                 preferred_element_type=jnp.float32) * (1.0 / math.sqrt(DE))
    ht = jax.nn.silu(ht)
    wt = jnp.dot(wm2t_ref[...], ht,
                 preferred_element_type=jnp.float32) * (1.0 / math.sqrt(H))
    wt = wt * eat_ref[...]
    wz = wt.T
    z = xs_ref[...] * wz
    m = jnp.dot(z, w2_ref[...],
                preferred_element_type=jnp.float32) * (1.0 / math.sqrt(D))
    o_ref[...] = jax.nn.silu(m)


def _tc_edge(xs, eet, eat, Wm1T, Wm2T, W2, q):
    be = 3200
    nb = ES // be
    return pl.pallas_call(
        _edge_body,
        grid=(nb,),
        in_specs=[
            pl.BlockSpec((be, D), lambda i: (i, 0)),
            pl.BlockSpec((DE, be), lambda i: (0, i + q * nb)),
            pl.BlockSpec((1, be), lambda i: (0, i + q * nb)),
            pl.BlockSpec((H, DE), lambda i: (0, 0)),
            pl.BlockSpec((D, H), lambda i: (0, 0)),
            pl.BlockSpec((D, D), lambda i: (0, 0)),
        ],
        out_specs=pl.BlockSpec((be, D), lambda i: (i, 0)),
        out_shape=jax.ShapeDtypeStruct((ES, D), jnp.float32),
    )(xs, eet, eat, Wm1T, Wm2T, W2)


def _pre_body(nf_ref, na_ref, wut_ref, wsct_ref, u_ref, sc_ref):
    na = na_ref[...]
    u_ref[...] = jnp.dot(na, wut_ref[...],
                         preferred_element_type=jnp.float32) * (
        1.0 / math.sqrt(DA))
    nf = nf_ref[...]
    sc = jnp.zeros((nf.shape[0], D), jnp.float32)
    for v in range(DA):
        wv = wsct_ref[pl.ds(v * D, D), :]
        sc = sc + na[:, v:v + 1] * jnp.dot(nf, wv,
                                           preferred_element_type=jnp.float32)
    sc_ref[...] = sc * (1.0 / math.sqrt(D * DA))


def _tc_pre(node_feats, node_attrs, WuT, WscT):
    bn = 2000
    return pl.pallas_call(
        _pre_body,
        grid=(N // bn,),
        in_specs=[
            pl.BlockSpec((bn, D), lambda i: (i, 0)),
            pl.BlockSpec((bn, DA), lambda i: (i, 0)),
            pl.BlockSpec((DA, D), lambda i: (0, 0)),
            pl.BlockSpec((DA * D, D), lambda i: (0, 0)),
        ],
        out_specs=[
            pl.BlockSpec((bn, D), lambda i: (i, 0)),
            pl.BlockSpec((bn, D), lambda i: (i, 0)),
        ],
        out_shape=[
            jax.ShapeDtypeStruct((N, D), jnp.float32),
            jax.ShapeDtypeStruct((N, D), jnp.float32),
        ],
    )(node_feats, node_attrs, WuT, WscT)


def _post_body(pa_ref, pb_ref, u_ref, sc_ref, w3_ref, o_ref):
    agg = (pa_ref[0] + pa_ref[1] + pb_ref[0] + pb_ref[1]) * (
        1.0 / math.sqrt(AVG_NEIGH))
    upd = jnp.dot(agg * u_ref[...], w3_ref[...],
                  preferred_element_type=jnp.float32) * (1.0 / math.sqrt(D))
    o_ref[...] = jax.nn.silu(upd + sc_ref[...])


def _tc_post(partsA, partsB, u, sc, W3):
    bn = 2000
    return pl.pallas_call(
        _post_body,
        grid=(N // bn,),
        in_specs=[
            pl.BlockSpec((NC, bn, D), lambda i: (0, i, 0)),
            pl.BlockSpec((NC, bn, D), lambda i: (0, i, 0)),
            pl.BlockSpec((bn, D), lambda i: (i, 0)),
            pl.BlockSpec((bn, D), lambda i: (i, 0)),
            pl.BlockSpec((D, D), lambda i: (0, 0)),
        ],
        out_specs=pl.BlockSpec((bn, D), lambda i: (i, 0)),
        out_shape=jax.ShapeDtypeStruct((N, D), jnp.float32),
    )(partsA, partsB, u, sc, W3)


def kernel(node_feats, node_attrs, edge_embedding, edge_attrs, edge_index,
           W1, Wm1, Wm2, W2, Wu, W3, Wsc):
    ei = jnp.pad(edge_index, ((0, 0), (0, CHUNK)))
    src3 = ei[0].reshape(NCHUNK + 1, 1, CHUNK)
    dst3 = ei[1].reshape(NCHUNK + 1, 1, CHUNK)
    eet = edge_embedding.T
    eat = edge_attrs.T
    Wm1T = Wm1.T
    Wm2T = Wm2.T
    WuT = Wu.T
    WscT = Wsc.transpose(1, 0, 2).reshape(DA * D, D)

    gathers, scatters = _sc_kernels()
    zeros = jnp.zeros((N, D), jnp.float32)
    x = _tc_x(node_feats, W1)
    msgs = []
    partsA = None
    for q in range(NSLICE):
        xs_q = gathers[q](x, src3)
        msgs.append(_tc_edge(xs_q, eet, eat, Wm1T, Wm2T, W2, q))
        if q == 1:
            partsA = scatters[0](msgs[0], msgs[1], dst3, zeros)
    partsB = scatters[1](msgs[2], msgs[3], dst3, zeros)
    u, sc = _tc_pre(node_feats, node_attrs, WuT, WscT)
    return _tc_post(partsA, partsB, u, sc, W3)
